# baseline (speedup 1.0000x reference)
import jax
import jax.numpy as jnp
from jax import lax
from jax.experimental import pallas as pl
from jax.experimental.pallas import tpu as pltpu
from jax.experimental.pallas import tpu_sc as plsc

N_OUT = 50000
N_IN = 12500
B = 4
C = 128
NW = 32
R = 64
E = 3 * R
NCHUNK = (B * N_OUT) // R
CH_MAX = -(-NCHUNK // NW)
FULL_W = NCHUNK - (CH_MAX - 1) * NW
TBL = CH_MAX * E
LANES = 16
GROUPS = R // LANES
CSL = C // LANES


def _body(x_hbm, idx_hbm, val_hbm, out_hbm,
          idxs_v, vals_v, g_v, o_v, gsem0, gsem1):
    cid = lax.axis_index("c")
    sid = lax.axis_index("s")
    wid = sid * 2 + cid
    nch = jnp.where(wid < FULL_W, CH_MAX, CH_MAX - 1)
    c0 = wid * CH_MAX - jnp.maximum(wid - FULL_W, 0)
    gsems = (gsem0, gsem1)

    pltpu.sync_copy(idx_hbm.at[pl.ds(c0 * E, TBL)], idxs_v)
    pltpu.sync_copy(val_hbm.at[pl.ds(c0 * E, TBL)], vals_v)

    def gather_refs(t, buf):
        for half in range(2):
            yield (
                x_hbm.at[idxs_v.at[pl.ds(t * E + half * (E // 2), E // 2)]],
                g_v.at[buf, pl.ds(half * (E // 2), E // 2)],
                gsems[buf],
            )

    def start_gather(t, buf):
        for src, dst, sem in gather_refs(t, buf):
            pltpu.async_copy(src, dst, sem)

    def wait_gather(t, buf):
        for src, dst, sem in gather_refs(t, buf):
            pltpu.make_async_copy(src, dst, sem).wait()

    def compute_write(t, buf):
        def group(q, _):
            wv = [
                vals_v[pl.ds(t * E + q * 3 * LANES + u * LANES, LANES)]
                for u in range(3)
            ]
            for k in range(LANES):
                i = q * LANES + k
                w = [wv[(3 * k + j) // LANES][(3 * k + j) % LANES]
                     for j in range(3)]
                for c in range(CSL):
                    sl = pl.ds(c * LANES, LANES)
                    o_v[buf, i, sl] = (
                        g_v[buf, 3 * i, sl] * w[0]
                        + g_v[buf, 3 * i + 1, sl] * w[1]
                        + g_v[buf, 3 * i + 2, sl] * w[2]
                    )
            return _

        lax.fori_loop(0, GROUPS, group, 0)
        pltpu.sync_copy(o_v.at[buf], out_hbm.at[pl.ds((c0 + t) * R, R)])

    start_gather(0, 0)

    def pair(p, _):
        for b in range(2):
            t = 2 * p + b
            tn = t + 1

            @pl.when(tn < nch)
            def _prefetch():
                start_gather(tn, 1 - b)

            @pl.when(t < nch)
            def _do():
                wait_gather(t, b)
                compute_write(t, b)
        return _

    lax.fori_loop(0, CH_MAX // 2, pair, 0)


@jax.jit
def _run(x2, idx_flat, val_flat):
    mesh = plsc.VectorSubcoreMesh(core_axis_name="c", subcore_axis_name="s")
    f = pl.kernel(
        _body,
        out_type=jax.ShapeDtypeStruct((B * N_OUT, C), jnp.float32),
        mesh=mesh,
        scratch_types=[
            pltpu.VMEM((TBL,), jnp.int32),
            pltpu.VMEM((TBL,), jnp.float32),
            pltpu.VMEM((2, E, C), jnp.float32),
            pltpu.VMEM((2, R, C), jnp.float32),
            pltpu.SemaphoreType.DMA,
            pltpu.SemaphoreType.DMA,
        ],
    )
    return f(x2, idx_flat, val_flat)


def kernel(x, index, value):
    col = index[1]
    npad = NW * TBL - B * 3 * N_OUT
    idx_flat = jnp.zeros(NW * TBL, jnp.int32).at[: B * 3 * N_OUT].set(
        (
            col.reshape(1, 3 * N_OUT)
            + (jnp.arange(B, dtype=jnp.int32) * N_IN).reshape(B, 1)
        ).reshape(B * 3 * N_OUT)
    )
    val_flat = jnp.zeros(NW * TBL, jnp.float32).at[: B * 3 * N_OUT].set(
        jnp.broadcast_to(value.reshape(1, 3 * N_OUT), (B, 3 * N_OUT)).reshape(
            B * 3 * N_OUT
        )
    )
    x2 = x.reshape(B * N_IN, C)
    out2 = _run(x2, idx_flat, val_flat)
    return out2.reshape(B, N_OUT, C)
